# baseline (speedup 1.0000x reference)
import functools

import jax
import jax.numpy as jnp
from jax import lax
from jax.experimental import pallas as pl
from jax.experimental.pallas import tpu as pltpu
from jax.experimental.pallas import tpu_sc as plsc

_K = 32
_MASK31 = 0x7FFFFFFF
_INT_MIN = -2147483648
_BIG = 3.0e38
_NW = 32
_RPW = 128 // _NW


def _keys_of(x):
    i = lax.bitcast_convert_type(x, jnp.int32)
    return jnp.where(i >= 0, i, i ^ jnp.int32(_MASK31))


def _kth_largest_key(key, k):
    rows = key.shape[0]
    tu = jnp.zeros((rows, 1), jnp.int32)

    def body(b, tu):
        bit = lax.shift_left(jnp.int32(1), 31 - b)
        cand = tu | bit
        thr = cand ^ jnp.int32(_INT_MIN)
        cnt = jnp.sum(jnp.where(key >= thr, 1.0, 0.0).astype(jnp.float32),
                      axis=1, keepdims=True)
        return jnp.where(cnt >= jnp.float32(k), cand, tu)

    tu = lax.fori_loop(0, 32, body, tu, unroll=True)
    return tu ^ jnp.int32(_INT_MIN)


def _key_to_float(kk):
    bits = jnp.where(kk >= 0, kk, kk ^ jnp.int32(_MASK31))
    return lax.bitcast_convert_type(bits, jnp.float32)


_G = 128
_NG = 8192 // _G


def _tc_a_body(x_ref, sr_ref, l16_ref):
    x = x_ref[...]
    m = jnp.max(x.reshape(128, _NG, _G), axis=2)
    lkey = _kth_largest_key(_keys_of(m), _K)
    l = _key_to_float(lkey)
    l16_ref[...] = jnp.broadcast_to(l, (128, 16))
    flags = jnp.where(m > l, 1.0, 0.0).astype(jnp.float32)
    ri = lax.broadcasted_iota(jnp.int32, (_NG, _NG), 0)
    ci = lax.broadcasted_iota(jnp.int32, (_NG, _NG), 1)
    tri = (ri <= ci).astype(jnp.float32)
    ranks = lax.dot_general(flags, tri, (((1,), (0,)), ((), ())),
                            preferred_element_type=jnp.float32)
    sr_ref[...] = flags * ranks


def _sc_body(sr_hbm, xt_hbm, buf_hbm, sr_v, idx_v, rows_v, sem):
    wid = lax.axis_index("s") * 2 + lax.axis_index("c")
    r0 = wid * _RPW
    pltpu.sync_copy(sr_hbm.at[pl.ds(r0 * _NG, _RPW * _NG)], sr_v)

    zeros = jnp.zeros((16,), jnp.int32)
    for v in range(8):
        idx_v[pl.ds(v * 16, 16)] = zeros

    lane = lax.iota(jnp.int32, 16)
    for row_i in range(_RPW):
        base = jnp.int32(row_i * _K - 1)
        for j in range(_NG // 16):
            rvec = sr_v[pl.ds(row_i * _NG + j * 16, 16)]
            flag = rvec > jnp.float32(0.0)
            pos = base + rvec.astype(jnp.int32)
            ids = lane + ((r0 + row_i) * _NG + j * 16)
            plsc.store_scatter(idx_v, [pos], ids, mask=flag)
    pltpu.async_copy(xt_hbm.at[idx_v], rows_v, sem).wait()
    pltpu.sync_copy(rows_v, buf_hbm.at[pl.ds(wid * 128, 128)])


def _sc_gather(sr_flat, xt):
    sc = pl.kernel(
        _sc_body,
        out_type=jax.ShapeDtypeStruct((4096, _G), jnp.float32),
        mesh=plsc.VectorSubcoreMesh(core_axis_name="c", subcore_axis_name="s"),
        compiler_params=pltpu.CompilerParams(needs_layout_passes=False),
        scratch_types=[
            pltpu.VMEM((_RPW * _NG,), jnp.float32),
            pltpu.VMEM((128,), jnp.int32),
            pltpu.VMEM((128, _G), jnp.float32),
            pltpu.SemaphoreType.DMA,
        ],
    )
    return sc(sr_flat, xt)


def _tc_b_body(x_ref, sr_ref, l16_ref, buf_ref, o_ref):
    x = x_ref[...]
    l1 = l16_ref[...][:, 0:1]
    n1 = jnp.max(sr_ref[...], axis=1, keepdims=True)
    buf = buf_ref[...].reshape(128, _K, _G)
    jf = lax.broadcasted_iota(jnp.int32, (1, _K, 1), 1).astype(jnp.float32)
    empty = jnp.where(jf >= n1[:, :, None], 1.0, 0.0)
    buf = buf - empty * jnp.float32(_BIG)
    tkey = _kth_largest_key(_keys_of(buf.reshape(128, _K * _G)), _K)
    kth = _key_to_float(jnp.maximum(_keys_of(l1), tkey))
    o_ref[...] = jnp.where(x >= kth, x, jnp.float32(0.0))


def kernel(x):
    sr, l16 = pl.pallas_call(
        _tc_a_body,
        out_shape=[jax.ShapeDtypeStruct((128, _NG), jnp.float32),
                   jax.ShapeDtypeStruct((128, 16), jnp.float32)],
    )(x)
    buf = _sc_gather(sr.reshape(128 * _NG), x.reshape(128 * _NG, _G))
    return pl.pallas_call(
        _tc_b_body,
        out_shape=jax.ShapeDtypeStruct(x.shape, x.dtype),
    )(x, sr, l16, buf)


# speedup vs baseline: 1.5904x; 1.5904x over previous
import jax
import jax.numpy as jnp
from jax import lax
from jax.experimental import pallas as pl
from jax.experimental.pallas import tpu as pltpu

_K = 32
_MASK31 = 0x7FFFFFFF
_INT_MIN = -2147483648
_BIG = 3.0e38
_BLK = 1024
_NB = 8192 // _BLK


def _keys_of(x):
    i = lax.bitcast_convert_type(x, jnp.int32)
    return jnp.where(i >= 0, i, i ^ jnp.int32(_MASK31))


def _kth_largest_key(key, k):
    rows = key.shape[0]
    tu = jnp.zeros((rows, 1), jnp.int32)

    def body(b, tu):
        bit = lax.shift_left(jnp.int32(1), 31 - b)
        cand = tu | bit
        thr = cand ^ jnp.int32(_INT_MIN)
        cnt = jnp.sum(jnp.where(key >= thr, 1.0, 0.0).astype(jnp.float32),
                      axis=1, keepdims=True)
        return jnp.where(cnt >= jnp.float32(k), cand, tu)

    tu = lax.fori_loop(0, 32, body, tu, unroll=True)
    return tu ^ jnp.int32(_INT_MIN)


def _key_to_float(kk):
    bits = jnp.where(kk >= 0, kk, kk ^ jnp.int32(_MASK31))
    return lax.bitcast_convert_type(bits, jnp.float32)


def _k1_body(x_ref, kth_ref, m_scr, sel_scr, buf_scr, lk_scr, ns_scr):
    p = pl.program_id(0)
    b = pl.program_id(1)
    blk = x_ref[...]

    @pl.when(jnp.logical_and(p == 0, b == 0))
    def _init():
        m_scr[...] = jnp.full((128, 256), _INT_MIN, jnp.int32)

    @pl.when(p == 0)
    def _maxes():
        k = _keys_of(blk)
        m = jnp.maximum(jnp.maximum(k[:, 0:256], k[:, 256:512]),
                        jnp.maximum(k[:, 512:768], k[:, 768:1024]))
        m_scr[...] = jnp.maximum(m_scr[...], m)

    @pl.when(jnp.logical_and(p == 1, b == 0))
    def _select():
        mk = m_scr[...]
        lkey = _kth_largest_key(mk, _K)
        lk_scr[...] = jnp.broadcast_to(lkey, (128, 128))
        flags = jnp.where(mk > lkey, 1.0, 0.0).astype(jnp.float32)
        ri = lax.broadcasted_iota(jnp.int32, (256, 256), 0)
        ci = lax.broadcasted_iota(jnp.int32, (256, 256), 1)
        tri = (ri <= ci).astype(jnp.float32)
        ranks = lax.dot_general(flags, tri, (((1,), (0,)), ((), ())),
                                preferred_element_type=jnp.float32)
        ns_scr[...] = jnp.broadcast_to(ranks[:, 255:256], (128, 128))
        ranks_i = ranks.astype(jnp.int32)
        jj = lax.broadcasted_iota(jnp.int32, (1, _K, 1), 1) + 1
        sel_scr[...] = jnp.where(
            (ranks_i[:, None, :] == jj) & (flags[:, None, :] > 0.0),
            1.0, 0.0).astype(jnp.float32)

    @pl.when(p == 1)
    def _compact():
        x3b = blk.reshape(128, 4, 256)
        contrib = lax.dot_general(
            x3b, sel_scr[...], (((2,), (2,)), ((0,), (0,))),
            precision=lax.Precision.HIGHEST,
            preferred_element_type=jnp.float32)
        buf_scr[:, pl.ds(b * 128, 128)] = contrib.reshape(128, 128)

    @pl.when(jnp.logical_and(p == 1, b == _NB - 1))
    def _finish():
        buf = buf_scr[...]
        col = lax.broadcasted_iota(jnp.int32, (1, 1024), 1)
        jcol = (col & (_K - 1)).astype(jnp.float32)
        empty = jnp.where(jcol >= ns_scr[...][:, 0:1], 1.0, 0.0)
        buf = buf - empty * jnp.float32(_BIG)
        tkey = _kth_largest_key(_keys_of(buf), _K)
        kth_key = jnp.maximum(lk_scr[...][:, 0:1], tkey)
        kth_ref[...] = jnp.broadcast_to(_key_to_float(kth_key), (128, 128))


def _k2_body(x_ref, kth_ref, o_ref):
    x = x_ref[...]
    kth = kth_ref[...][:, 0:1]
    o_ref[...] = jnp.where(x >= kth, x, jnp.float32(0.0))


def kernel(x):
    kth = pl.pallas_call(
        _k1_body,
        grid=(2, _NB),
        in_specs=[pl.BlockSpec((128, _BLK), lambda p, b: (0, b))],
        out_specs=pl.BlockSpec((128, 128), lambda p, b: (0, 0)),
        out_shape=jax.ShapeDtypeStruct((128, 128), jnp.float32),
        scratch_shapes=[
            pltpu.VMEM((128, 256), jnp.int32),
            pltpu.VMEM((128, _K, 256), jnp.float32),
            pltpu.VMEM((128, 1024), jnp.float32),
            pltpu.VMEM((128, 128), jnp.int32),
            pltpu.VMEM((128, 128), jnp.float32),
        ],
    )(x)
    return pl.pallas_call(
        _k2_body,
        grid=(_NB,),
        in_specs=[pl.BlockSpec((128, _BLK), lambda b: (0, b)),
                  pl.BlockSpec((128, 128), lambda b: (0, 0))],
        out_specs=pl.BlockSpec((128, _BLK), lambda b: (0, b)),
        out_shape=jax.ShapeDtypeStruct(x.shape, x.dtype),
    )(x, kth)


# speedup vs baseline: 2.1752x; 1.3677x over previous
import jax
import jax.numpy as jnp
from jax import lax
from jax.experimental import pallas as pl
from jax.experimental.pallas import tpu as pltpu

_K = 32
_MASK31 = 0x7FFFFFFF
_INT_MIN = -2147483648
_BIG = 3.0e38
_BLK = 1024
_NB = 8192 // _BLK


def _keys_of(x):
    i = lax.bitcast_convert_type(x, jnp.int32)
    return jnp.where(i >= 0, i, i ^ jnp.int32(_MASK31))


def _kth_largest_key(key, k):
    rows = key.shape[0]
    tu = jnp.zeros((rows, 1), jnp.int32)

    def body(b, tu):
        bit = lax.shift_left(jnp.int32(1), 31 - b)
        cand = tu | bit
        thr = cand ^ jnp.int32(_INT_MIN)
        cnt = jnp.sum(jnp.where(key >= thr, 1.0, 0.0).astype(jnp.float32),
                      axis=1, keepdims=True)
        return jnp.where(cnt >= jnp.float32(k), cand, tu)

    tu = lax.fori_loop(0, 32, body, tu, unroll=True)
    return tu ^ jnp.int32(_INT_MIN)


def _key_to_float(kk):
    bits = jnp.where(kk >= 0, kk, kk ^ jnp.int32(_MASK31))
    return lax.bitcast_convert_type(bits, jnp.float32)


def _k1_body(x_ref, kth_ref, m_scr, sel_scr, buf_scr, lk_scr, ns_scr):
    p = pl.program_id(0)
    b = pl.program_id(1)
    blk = x_ref[...]

    @pl.when(jnp.logical_and(p == 0, b == 0))
    def _init():
        m_scr[...] = jnp.full((128, 256), _INT_MIN, jnp.int32)

    @pl.when(p == 0)
    def _maxes():
        k = _keys_of(blk)
        m = jnp.maximum(jnp.maximum(k[:, 0:256], k[:, 256:512]),
                        jnp.maximum(k[:, 512:768], k[:, 768:1024]))
        m_scr[...] = jnp.maximum(m_scr[...], m)

    @pl.when(jnp.logical_and(p == 1, b == 0))
    def _select():
        mk = m_scr[...]
        lkey = _kth_largest_key(mk, _K)
        lk_scr[...] = jnp.broadcast_to(lkey, (128, 128))
        flags = jnp.where(mk > lkey, 1.0, 0.0).astype(jnp.float32)
        ri = lax.broadcasted_iota(jnp.int32, (256, 256), 0)
        ci = lax.broadcasted_iota(jnp.int32, (256, 256), 1)
        tri = (ri <= ci).astype(jnp.float32)
        ranks = lax.dot_general(flags, tri, (((1,), (0,)), ((), ())),
                                preferred_element_type=jnp.float32)
        ns_scr[...] = jnp.broadcast_to(ranks[:, 255:256], (128, 128))
        ranks_i = ranks.astype(jnp.int32)
        jj = lax.broadcasted_iota(jnp.int32, (1, _K, 1), 1) + 1
        sel_scr[...] = jnp.where(
            (ranks_i[:, None, :] == jj) & (flags[:, None, :] > 0.0),
            1.0, 0.0).astype(jnp.bfloat16)

    @pl.when(p == 1)
    def _compact():
        hi = blk.astype(jnp.bfloat16)
        r1 = blk - hi.astype(jnp.float32)
        mid = r1.astype(jnp.bfloat16)
        lo = (r1 - mid.astype(jnp.float32)).astype(jnp.bfloat16)
        sel = sel_scr[...]

        def dot_part(part):
            p3 = part.reshape(128, 4, 256)
            return lax.dot_general(
                p3, sel, (((2,), (2,)), ((0,), (0,))),
                preferred_element_type=jnp.float32)

        contrib = dot_part(hi) + dot_part(mid) + dot_part(lo)
        buf_scr[:, pl.ds(b * 128, 128)] = contrib.reshape(128, 128)

    @pl.when(jnp.logical_and(p == 1, b == _NB - 1))
    def _finish():
        buf = buf_scr[...]
        col = lax.broadcasted_iota(jnp.int32, (1, 1024), 1)
        jcol = (col & (_K - 1)).astype(jnp.float32)
        empty = jnp.where(jcol >= ns_scr[...][:, 0:1], 1.0, 0.0)
        buf = buf - empty * jnp.float32(_BIG)
        tkey = _kth_largest_key(_keys_of(buf), _K)
        kth_key = jnp.maximum(lk_scr[...][:, 0:1], tkey)
        kth_ref[...] = jnp.broadcast_to(_key_to_float(kth_key), (128, 128))


def _k2_body(x_ref, kth_ref, o_ref):
    x = x_ref[...]
    kth = kth_ref[...][:, 0:1]
    o_ref[...] = jnp.where(x >= kth, x, jnp.float32(0.0))


def kernel(x):
    kth = pl.pallas_call(
        _k1_body,
        grid=(2, _NB),
        in_specs=[pl.BlockSpec((128, _BLK), lambda p, b: (0, b))],
        out_specs=pl.BlockSpec((128, 128), lambda p, b: (0, 0)),
        out_shape=jax.ShapeDtypeStruct((128, 128), jnp.float32),
        scratch_shapes=[
            pltpu.VMEM((128, 256), jnp.int32),
            pltpu.VMEM((128, _K, 256), jnp.bfloat16),
            pltpu.VMEM((128, 1024), jnp.float32),
            pltpu.VMEM((128, 128), jnp.int32),
            pltpu.VMEM((128, 128), jnp.float32),
        ],
    )(x)
    return pl.pallas_call(
        _k2_body,
        grid=(_NB,),
        in_specs=[pl.BlockSpec((128, _BLK), lambda b: (0, b)),
                  pl.BlockSpec((128, 128), lambda b: (0, 0))],
        out_specs=pl.BlockSpec((128, _BLK), lambda b: (0, b)),
        out_shape=jax.ShapeDtypeStruct(x.shape, x.dtype),
    )(x, kth)


# speedup vs baseline: 3.1125x; 1.4309x over previous
import jax
import jax.numpy as jnp
from jax import lax
from jax.experimental import pallas as pl
from jax.experimental.pallas import tpu as pltpu

_K = 32
_MASK31 = 0x7FFFFFFF
_INT_MIN = -2147483648
_BIG = 3.0e38
_BLK = 1024
_NB = 8192 // _BLK


def _keys_of(x):
    i = lax.bitcast_convert_type(x, jnp.int32)
    return jnp.where(i >= 0, i, i ^ jnp.int32(_MASK31))


def _kth_largest_key(key, k):
    rows = key.shape[0]
    tu = jnp.zeros((rows, 1), jnp.int32)

    def body(b, tu):
        bit = lax.shift_left(jnp.int32(1), 31 - b)
        cand = tu | bit
        thr = cand ^ jnp.int32(_INT_MIN)
        cnt = jnp.sum(jnp.where(key >= thr, 1.0, 0.0).astype(jnp.float32),
                      axis=1, keepdims=True)
        return jnp.where(cnt >= jnp.float32(k), cand, tu)

    tu = lax.fori_loop(0, 32, body, tu, unroll=True)
    return tu ^ jnp.int32(_INT_MIN)


def _key_to_float(kk):
    bits = jnp.where(kk >= 0, kk, kk ^ jnp.int32(_MASK31))
    return lax.bitcast_convert_type(bits, jnp.float32)


def _k1_body(xb_ref, xf_ref, kth_ref, m_scr):
    i = pl.program_id(0)
    blk = xb_ref[...]

    @pl.when(i == 0)
    def _init():
        m_scr[...] = jnp.full((128, 256), _INT_MIN, jnp.int32)

    @pl.when(i < _NB)
    def _maxes():
        k = _keys_of(blk)
        m = jnp.maximum(jnp.maximum(k[:, 0:256], k[:, 256:512]),
                        jnp.maximum(k[:, 512:768], k[:, 768:1024]))
        m_scr[...] = jnp.maximum(m_scr[...], m)

    @pl.when(i == _NB)
    def _finish():
        x = xf_ref[...]
        mk = m_scr[...]
        lkey = _kth_largest_key(mk, _K)

        flags = jnp.where(mk > lkey, 1.0, 0.0).astype(jnp.float32)
        ri = lax.broadcasted_iota(jnp.int32, (256, 256), 0)
        ci = lax.broadcasted_iota(jnp.int32, (256, 256), 1)
        tri = (ri <= ci).astype(jnp.float32)
        ranks = lax.dot_general(flags, tri, (((1,), (0,)), ((), ())),
                                preferred_element_type=jnp.float32)

        ranks_i = ranks.astype(jnp.int32)
        jj = lax.broadcasted_iota(jnp.int32, (1, _K, 1), 1) + 1
        sel = jnp.where(
            (ranks_i[:, None, :] == jj) & (flags[:, None, :] > 0.0),
            1.0, 0.0).astype(jnp.float32)

        x3 = x.reshape(128, 32, 256)
        buf = lax.dot_general(
            sel, x3, (((2,), (2,)), ((0,), (0,))),
            precision=lax.Precision.HIGHEST,
            preferred_element_type=jnp.float32)
        nsel = ranks[:, 255:256]
        jf = lax.broadcasted_iota(jnp.int32, (1, _K, 1), 1).astype(jnp.float32)
        empty = (jf >= nsel[:, :, None]).astype(jnp.float32)
        buf = buf - empty * jnp.float32(_BIG)

        tkey = _kth_largest_key(_keys_of(buf.reshape(128, _K * _K)), _K)
        kth_key = jnp.maximum(lkey, tkey)
        kth_ref[...] = jnp.broadcast_to(_key_to_float(kth_key), (128, 128))


def _k2_body(x_ref, kth_ref, o_ref):
    x = x_ref[...]
    kth = kth_ref[...][:, 0:1]
    o_ref[...] = jnp.where(x >= kth, x, jnp.float32(0.0))


def kernel(x):
    kth = pl.pallas_call(
        _k1_body,
        grid=(_NB + 1,),
        in_specs=[pl.BlockSpec((128, _BLK), lambda i: (0, jnp.minimum(i, _NB - 1))),
                  pl.BlockSpec((128, 8192), lambda i: (0, 0))],
        out_specs=pl.BlockSpec((128, 128), lambda i: (0, 0)),
        out_shape=jax.ShapeDtypeStruct((128, 128), jnp.float32),
        scratch_shapes=[pltpu.VMEM((128, 256), jnp.int32)],
    )(x, x)
    return pl.pallas_call(
        _k2_body,
        grid=(_NB,),
        in_specs=[pl.BlockSpec((128, _BLK), lambda b: (0, b)),
                  pl.BlockSpec((128, 128), lambda b: (0, 0))],
        out_specs=pl.BlockSpec((128, _BLK), lambda b: (0, b)),
        out_shape=jax.ShapeDtypeStruct(x.shape, x.dtype),
    )(x, kth)


# speedup vs baseline: 5.3906x; 1.7319x over previous
import jax
import jax.numpy as jnp
from jax.experimental import pallas as pl
from jax.experimental.pallas import tpu as pltpu

_K = 32
_MASK31 = 0x7FFFFFFF
_INT_MIN = -2147483648
_BIG = 3.0e38


def _keys_of(x):
    i = jax.lax.bitcast_convert_type(x, jnp.int32)
    return jnp.where(i >= 0, i, i ^ jnp.int32(_MASK31))


def _kth_largest_key(key, k):
    rows = key.shape[0]
    tu = jnp.zeros((rows, 1), jnp.int32)

    def body(s, tu):
        sh = 30 - 2 * s
        digit = jnp.zeros((rows, 1), jnp.int32)
        for c in (1, 2, 3):
            cand = tu | jax.lax.shift_left(jnp.int32(c), sh)
            thr = cand ^ jnp.int32(_INT_MIN)
            cnt = jnp.sum(jnp.where(key >= thr, 1.0, 0.0).astype(jnp.float32),
                          axis=1, keepdims=True)
            digit = digit + jnp.where(cnt >= jnp.float32(k), 1, 0)
        return tu | jax.lax.shift_left(digit, sh)

    tu = jax.lax.fori_loop(0, 16, body, tu, unroll=True)
    return tu ^ jnp.int32(_INT_MIN)


def _sparsify_body(x_ref, o_ref):
    x = x_ref[...]
    key = _keys_of(x)

    m = key[:, 0:256]
    for e in range(1, 32):
        m = jnp.maximum(m, key[:, e * 256:(e + 1) * 256])

    lkey = _kth_largest_key(m, _K)

    flags = (m > lkey).astype(jnp.float32)
    ri = jax.lax.broadcasted_iota(jnp.int32, (256, 256), 0)
    ci = jax.lax.broadcasted_iota(jnp.int32, (256, 256), 1)
    tri = (ri <= ci).astype(jnp.float32)
    ranks = jax.lax.dot_general(
        flags, tri, (((1,), (0,)), ((), ())),
        preferred_element_type=jnp.float32)

    ranks_i = ranks.astype(jnp.int32)
    jj = jax.lax.broadcasted_iota(jnp.int32, (1, _K, 1), 1) + 1
    sel = jnp.where(
        (ranks_i[:, None, :] == jj) & (flags[:, None, :] > 0.0),
        1.0, 0.0).astype(jnp.float32)

    x3 = x.reshape(128, 32, 256)
    buf = jax.lax.dot_general(
        sel, x3, (((2,), (2,)), ((0,), (0,))),
        precision=jax.lax.Precision.HIGHEST,
        preferred_element_type=jnp.float32)
    nsel = ranks[:, 255:256]
    jf = jax.lax.broadcasted_iota(jnp.int32, (1, _K, 1), 1).astype(jnp.float32)
    empty = (jf >= nsel[:, :, None]).astype(jnp.float32)
    buf = buf - empty * jnp.float32(_BIG)

    bkey = _keys_of(buf.reshape(128, 32 * _K))
    tkey = _kth_largest_key(bkey, _K)

    kth_key = jnp.maximum(lkey, tkey)
    kth_bits = jnp.where(kth_key >= 0, kth_key, kth_key ^ jnp.int32(_MASK31))
    kth = jax.lax.bitcast_convert_type(kth_bits, jnp.float32)
    o_ref[...] = jnp.where(x >= kth, x, jnp.float32(0.0))


def kernel(x):
    return pl.pallas_call(
        _sparsify_body,
        out_shape=jax.ShapeDtypeStruct(x.shape, x.dtype),
    )(x)


# speedup vs baseline: 5.6048x; 1.0397x over previous
import jax
import jax.numpy as jnp
from jax.experimental import pallas as pl
from jax.experimental.pallas import tpu as pltpu

_K = 32
_MASK31 = 0x7FFFFFFF
_INT_MIN = -2147483648
_BIG = 3.0e38


def _keys_of(x):
    i = jax.lax.bitcast_convert_type(x, jnp.int32)
    return jnp.where(i >= 0, i, i ^ jnp.int32(_MASK31))


def _kth_largest_key(key, k):
    rows = key.shape[0]
    tu = jnp.zeros((rows, 1), jnp.int32)

    def body(s, tu):
        sh = 30 - 2 * s
        digit = jnp.zeros((rows, 1), jnp.int32)
        for c in (1, 2, 3):
            cand = tu | jax.lax.shift_left(jnp.int32(c), sh)
            thr = cand ^ jnp.int32(_INT_MIN)
            cnt = jnp.sum(jnp.where(key >= thr, 1.0, 0.0).astype(jnp.float32),
                          axis=1, keepdims=True)
            digit = digit + jnp.where(cnt >= jnp.float32(k), 1, 0)
        return tu | jax.lax.shift_left(digit, sh)

    tu = jax.lax.fori_loop(0, 16, body, tu, unroll=True)
    return tu ^ jnp.int32(_INT_MIN)


def _sparsify_body(x_ref, o_ref):
    x = x_ref[...]
    key = _keys_of(x)

    m = key[:, 0:256]
    for e in range(1, 32):
        m = jnp.maximum(m, key[:, e * 256:(e + 1) * 256])

    lkey = _kth_largest_key(m, _K)

    flags = (m > lkey).astype(jnp.float32)
    ri = jax.lax.broadcasted_iota(jnp.int32, (256, 256), 0)
    ci = jax.lax.broadcasted_iota(jnp.int32, (256, 256), 1)
    tri = (ri <= ci).astype(jnp.float32)
    ranks = jax.lax.dot_general(
        flags, tri, (((1,), (0,)), ((), ())),
        preferred_element_type=jnp.float32)

    fr = ranks * flags
    jj = (jax.lax.broadcasted_iota(jnp.int32, (1, _K, 1), 1) + 1
          ).astype(jnp.float32)
    sel = jnp.where(fr[:, None, :] == jj, 1.0, 0.0)

    x3 = x.reshape(128, 32, 256)
    buf = jax.lax.dot_general(
        sel, x3, (((2,), (2,)), ((0,), (0,))),
        precision=jax.lax.Precision.HIGHEST,
        preferred_element_type=jnp.float32)
    nsel = ranks[:, 255:256]
    jf = jax.lax.broadcasted_iota(jnp.int32, (1, _K, 1), 1).astype(jnp.float32)
    empty = (jf >= nsel[:, :, None]).astype(jnp.float32)
    buf = buf - empty * jnp.float32(_BIG)

    bkey = _keys_of(buf.reshape(128, 32 * _K))
    tkey = _kth_largest_key(bkey, _K)

    kth_key = jnp.maximum(lkey, tkey)
    kth_bits = jnp.where(kth_key >= 0, kth_key, kth_key ^ jnp.int32(_MASK31))
    kth = jax.lax.bitcast_convert_type(kth_bits, jnp.float32)
    o_ref[...] = jnp.where(x >= kth, x, jnp.float32(0.0))


def kernel(x):
    return pl.pallas_call(
        _sparsify_body,
        out_shape=jax.ShapeDtypeStruct(x.shape, x.dtype),
    )(x)
